# baseline (speedup 1.0000x reference)
import jax
import jax.numpy as jnp
from jax import lax
from jax.experimental import pallas as pl
from jax.experimental.pallas import tpu as pltpu
from jax.experimental.pallas import tpu_sc as plsc

NC, NS = 2, 16
NW = NC * NS
EDIM = 32
CTX = 20
NNEGS = 5
NEG = CTX * NNEGS
COLS = CTX + NEG
CHUNK = 128


def _gather_body(iw2d, aw3d, in_embed, out_embed, ei_out, rows_out,
                 ibuf, abuf, erbuf, rbuf, sem):
    cpw = aw3d.shape[1]
    wid = lax.axis_index("s") * NC + lax.axis_index("c")

    pltpu.sync_copy(iw2d.at[wid], ibuf)
    pltpu.async_copy(in_embed.at[ibuf], erbuf, sem).wait()
    pltpu.sync_copy(erbuf, ei_out.at[pl.ds(wid * CHUNK, CHUNK)])

    pltpu.sync_copy(aw3d.at[wid], abuf)
    base = wid * cpw * CHUNK
    pltpu.async_copy(out_embed.at[abuf.at[0]], rbuf.at[0], sem).wait()

    def chunk(j, carry):
        slot = lax.rem(j, 2)
        nxt = lax.rem(j + 1, 2)
        cp = pltpu.async_copy(out_embed.at[abuf.at[j + 1]], rbuf.at[nxt], sem)
        pltpu.sync_copy(rbuf.at[slot],
                        rows_out.at[pl.ds(base + j * CHUNK, CHUNK)])
        cp.wait()
        return carry

    lax.fori_loop(0, cpw - 1, chunk, 0)
    pltpu.sync_copy(rbuf.at[lax.rem(cpw - 1, 2)],
                    rows_out.at[pl.ds(base + (cpw - 1) * CHUNK, CHUNK)])


def _sc_gather(iword, allwords, in_embed, out_embed):
    B = iword.shape[0]
    cpw = (B * COLS) // (NW * CHUNK)
    mesh = plsc.VectorSubcoreMesh(core_axis_name="c", subcore_axis_name="s",
                                  num_cores=NC, num_subcores=NS)
    f = pl.kernel(
        _gather_body,
        out_type=(jax.ShapeDtypeStruct((B, EDIM), jnp.float32),
                  jax.ShapeDtypeStruct((B * COLS, EDIM), jnp.float32)),
        mesh=mesh,
        compiler_params=pltpu.CompilerParams(use_tc_tiling_on_sc=False),
        scratch_types=[
            pltpu.VMEM((CHUNK,), jnp.int32),
            pltpu.VMEM((cpw, CHUNK), jnp.int32),
            pltpu.VMEM((CHUNK, EDIM), jnp.float32),
            pltpu.VMEM((2, CHUNK, EDIM), jnp.float32),
            pltpu.SemaphoreType.DMA,
        ],
    )
    return f(iword.reshape(NW, B // NW),
             allwords.reshape(NW, cpw, CHUNK),
             in_embed, out_embed)


def _loss_body(rows_ref, ei_ref, out_ref):
    @pl.when(pl.program_id(0) == 0)
    def _():
        out_ref[...] = jnp.zeros((1, 1), jnp.float32)
    rows = rows_ref[...]
    ei = ei_ref[...]
    s = jnp.sum(rows * ei[:, None, :], axis=2)
    col = lax.broadcasted_iota(jnp.int32, s.shape, 1)
    x = jnp.where(col < CTX, s, -s)
    out_ref[...] -= jnp.reshape(jnp.sum(jnp.log(jax.nn.sigmoid(x))), (1, 1))


def kernel(iword, owords, nwords, in_embed, out_embed):
    B = iword.shape[0]
    iw = iword.astype(jnp.int32)
    aw = jnp.concatenate([owords.astype(jnp.int32),
                          nwords.astype(jnp.int32)], axis=1).reshape(B * COLS)
    ei_rows, orows = _sc_gather(iw, aw, in_embed, out_embed)
    TB = 128
    tot = pl.pallas_call(
        _loss_body,
        grid=(B // TB,),
        in_specs=[
            pl.BlockSpec((TB, COLS, EDIM), lambda i: (i, 0, 0)),
            pl.BlockSpec((TB, EDIM), lambda i: (i, 0)),
        ],
        out_specs=pl.BlockSpec((1, 1), lambda i: (0, 0)),
        out_shape=jax.ShapeDtypeStruct((1, 1), jnp.float32),
    )(orows.reshape(B, COLS, EDIM), ei_rows)
    return jnp.reshape(tot, ()) / (CTX * B)


# speedup vs baseline: 1.3524x; 1.3524x over previous
import jax
import jax.numpy as jnp
from jax import lax
from jax.experimental import pallas as pl
from jax.experimental.pallas import tpu as pltpu
from jax.experimental.pallas import tpu_sc as plsc

NC, NS = 2, 16
NW = NC * NS
EDIM = 32
CTX = 20
NNEGS = 5
NEG = CTX * NNEGS
COLS = CTX + NEG
GRP = 8
BPW = 128
NGRP = BPW // GRP
RPG = GRP * COLS


def _sc_body(iw2d, aw3d, in_embed, out_embed, scores,
             ibuf, abuf, eib0, eib1, rows0, rows1, accv, sem0, sem1):
    wid = lax.axis_index("s") * NC + lax.axis_index("c")
    base = wid * BPW

    pltpu.sync_copy(iw2d.at[wid], ibuf)
    pltpu.sync_copy(aw3d.at[wid], abuf)

    def fire(g, eib, rows, sem):
        b0 = g * GRP
        pltpu.async_copy(in_embed.at[ibuf.at[pl.ds(b0, GRP)]], eib, sem)
        for j in range(GRP):
            pltpu.async_copy(out_embed.at[abuf.at[b0 + j]],
                             rows.at[pl.ds(j * COLS, COLS)], sem)

    def drain(eib, rows, sem):
        pltpu.make_async_copy(in_embed.at[pl.ds(0, GRP)], eib, sem).wait()
        pltpu.make_async_copy(out_embed.at[pl.ds(0, RPG)], rows, sem).wait()

    lane = lax.iota(jnp.int32, 16)
    masks = [lane == j for j in range(16)]
    zv = jnp.zeros((16,), jnp.float32)

    def compute(g, eib, rows):
        def per_b(bb, c1):
            ei0 = eib[bb, pl.ds(0, 16)]
            ei1 = eib[bb, pl.ds(16, 16)]
            rb = bb * COLS

            def col_score(r):
                p = rows[r, pl.ds(0, 16)] * ei0 + rows[r, pl.ds(16, 16)] * ei1
                return jnp.sum(p)

            def per_chunk(ch, c2):
                r0 = rb + ch * 16
                vec = zv
                for j in range(16):
                    s = col_score(r0 + j)
                    vec = jnp.where(masks[j], jnp.full((16,), s), vec)
                accv[bb, pl.ds(ch * 16, 16)] = vec
                return c2
            lax.fori_loop(0, COLS // 16, per_chunk, 0)

            vec = zv
            for j in range(8):
                s = col_score(rb + 112 + j)
                vec = jnp.where(masks[j], jnp.full((16,), s), vec)
            accv[bb, pl.ds(112, 16)] = vec
            return c1
        lax.fori_loop(0, GRP, per_b, 0)
        pltpu.sync_copy(accv, scores.at[pl.ds(base + g * GRP, GRP)])

    fire(0, eib0, rows0, sem0)
    fire(1, eib1, rows1, sem1)

    def pair(p, carry):
        g0 = 2 * p
        drain(eib0, rows0, sem0)
        compute(g0, eib0, rows0)

        @pl.when(p < NGRP // 2 - 1)
        def _():
            fire(g0 + 2, eib0, rows0, sem0)
        drain(eib1, rows1, sem1)
        compute(g0 + 1, eib1, rows1)

        @pl.when(p < NGRP // 2 - 1)
        def _():
            fire(g0 + 3, eib1, rows1, sem1)
        return carry

    lax.fori_loop(0, NGRP // 2, pair, 0)


def _sc_scores(iword, allwords, in_embed, out_embed):
    B = iword.shape[0]
    mesh = plsc.VectorSubcoreMesh(core_axis_name="c", subcore_axis_name="s",
                                  num_cores=NC, num_subcores=NS)
    f = pl.kernel(
        _sc_body,
        out_type=jax.ShapeDtypeStruct((B, 128), jnp.float32),
        mesh=mesh,
        compiler_params=pltpu.CompilerParams(use_tc_tiling_on_sc=False,
                                             needs_layout_passes=False),
        scratch_types=[
            pltpu.VMEM((BPW,), jnp.int32),
            pltpu.VMEM((BPW, COLS), jnp.int32),
            pltpu.VMEM((GRP, EDIM), jnp.float32),
            pltpu.VMEM((GRP, EDIM), jnp.float32),
            pltpu.VMEM((RPG, EDIM), jnp.float32),
            pltpu.VMEM((RPG, EDIM), jnp.float32),
            pltpu.VMEM((GRP, 128), jnp.float32),
            pltpu.SemaphoreType.DMA,
            pltpu.SemaphoreType.DMA,
        ],
    )
    return f(iword.reshape(NW, BPW),
             allwords.reshape(NW, BPW, COLS),
             in_embed, out_embed)


def _loss_body(sc_ref, out_ref):
    x = sc_ref[...]
    col = lax.broadcasted_iota(jnp.int32, x.shape, 1)
    xs = jnp.where(col < CTX, x, -x)
    contrib = jnp.where(col < COLS, jnp.log(jax.nn.sigmoid(xs)), 0.0)
    out_ref[...] = jnp.reshape(-jnp.sum(contrib), (1, 1))


def kernel(iword, owords, nwords, in_embed, out_embed):
    B = iword.shape[0]
    iw = iword.astype(jnp.int32)
    aw = jnp.concatenate([owords.astype(jnp.int32),
                          nwords.astype(jnp.int32)], axis=1)
    scores = _sc_scores(iw, aw, in_embed, out_embed)
    tot = pl.pallas_call(
        _loss_body,
        out_shape=jax.ShapeDtypeStruct((1, 1), jnp.float32),
    )(scores)
    return jnp.reshape(tot, ()) / (CTX * B)
